# baseline (speedup 1.0000x reference)
import numpy as np

import jax
import jax.numpy as jnp
from jax.experimental import pallas as pl
from jax.experimental.pallas import tpu as pltpu

_EDGES = [(0, 1), (1, 2), (2, 3), (0, 4), (4, 5), (5, 6), (0, 7), (7, 8),
          (8, 9), (9, 10), (8, 11), (11, 12), (12, 13), (8, 14), (14, 15),
          (15, 16)]
_N = 17
_D = 64
_F = _N * _D
_TILE = 128
_NT = (_F + _TILE - 1) // _TILE


def _adj_structure():
    a = np.eye(_N, dtype=bool)
    for i, j in _EDGES:
        a[i, j] = True
        a[j, i] = True
    return a


def _tile_pairs():
    a = _adj_structure()
    pairs = []
    for j in range(_NT):
        ms = [m for m in (2 * j, 2 * j + 1) if m < _N]
        for jp in range(_NT):
            ns = [n for n in (2 * jp, 2 * jp + 1) if n < _N]
            if any(a[n, m] for m in ms for n in ns):
                pairs.append((j, jp))
    return pairs


_PAIRS = _tile_pairs()
_IN_TILES = [[j for (j, jp) in _PAIRS if jp == jpp] for jpp in range(_NT)]


def _sz(j):
    return min(_F - j * _TILE, _TILE)


def _conv_kernel(x_ref, t_ref, raw_ref, sum_ref, sq_ref):
    i = pl.program_id(0)

    @pl.when(i == 0)
    def _init():
        sum_ref[...] = jnp.zeros_like(sum_ref)
        sq_ref[...] = jnp.zeros_like(sq_ref)

    for jp in range(_NT):
        c0 = jp * _TILE
        acc = None
        for j in _IN_TILES[jp]:
            r0 = j * _TILE
            prod = jnp.dot(
                x_ref[:, r0:r0 + _sz(j)],
                t_ref[r0:r0 + _sz(j), c0:c0 + _sz(jp)],
                preferred_element_type=jnp.float32)
            acc = prod if acc is None else acc + prod
        raw_ref[:, c0:c0 + _sz(jp)] = acc
        sum_ref[:, c0:c0 + _sz(jp)] += jnp.sum(acc, axis=0, keepdims=True)
        sq_ref[:, c0:c0 + _sz(jp)] += jnp.sum(acc * acc, axis=0, keepdims=True)


def _bn_kernel(raw_ref, scale_ref, shift_ref, out_ref):
    out_ref[...] = jnp.maximum(
        raw_ref[...] * scale_ref[...] + shift_ref[...], 0.0)


def kernel(x, adj, W, b, gamma, beta):
    B, N, D = x.shape
    F = N * D
    xf = x.reshape(B, F)
    T = jnp.kron(adj.T, W)
    TB = 1024 if B % 1024 == 0 else B
    grid = (B // TB,)

    x_spec = pl.BlockSpec((TB, F), lambda i: (i, 0))
    t_spec = pl.BlockSpec((F, F), lambda i: (0, 0))
    vec_spec = pl.BlockSpec((1, F), lambda i: (0, 0))

    raw, sums, sq = pl.pallas_call(
        _conv_kernel,
        grid=grid,
        in_specs=[x_spec, t_spec],
        out_specs=[x_spec, vec_spec, vec_spec],
        out_shape=[
            jax.ShapeDtypeStruct((B, F), jnp.float32),
            jax.ShapeDtypeStruct((1, F), jnp.float32),
            jax.ShapeDtypeStruct((1, F), jnp.float32),
        ],
        compiler_params=pltpu.CompilerParams(
            dimension_semantics=("arbitrary",),
        ),
    )(xf, T)

    cnt = float(B * N)
    s_c = sums.reshape(N, D).sum(axis=0)
    q_c = sq.reshape(N, D).sum(axis=0)
    mean = s_c / cnt + b
    var = q_c / cnt - (s_c / cnt) ** 2
    scale = gamma * jax.lax.rsqrt(var + 1e-5)
    shift = (b - mean) * scale + beta
    scale_f = jnp.tile(scale, N).reshape(1, F)
    shift_f = jnp.tile(shift, N).reshape(1, F)

    out = pl.pallas_call(
        _bn_kernel,
        grid=grid,
        in_specs=[x_spec, vec_spec, vec_spec],
        out_specs=x_spec,
        out_shape=jax.ShapeDtypeStruct((B, F), jnp.float32),
        compiler_params=pltpu.CompilerParams(
            dimension_semantics=("parallel",),
        ),
    )(raw, scale_f, shift_f)
    return out.reshape(B, N, D)
